# baseline (speedup 1.0000x reference)
import functools

import jax
import jax.numpy as jnp
from jax import lax
from jax.experimental import pallas as pl
from jax.experimental.pallas import tpu as pltpu
from jax.experimental.pallas import tpu_sc as plsc

L = 16


def _tc_transpose(xt, V, D, BLK):
    def body(x_ref, o_ref):
        eye = jnp.eye(D, dtype=jnp.float32)
        h = BLK // 2
        e = jax.lax.dot_general(
            x_ref[:, :h], eye, (((0,), (0,)), ((), ())),
            preferred_element_type=jnp.float32)
        o = jax.lax.dot_general(
            x_ref[:, h:], eye, (((0,), (0,)), ((), ())),
            preferred_element_type=jnp.float32)
        o_ref[...] = jnp.concatenate([e, o], axis=1)

    nblk = pl.cdiv(V, BLK)
    packed = pl.pallas_call(
        body,
        grid=(nblk,),
        in_specs=[pl.BlockSpec((D, BLK), lambda b: (0, b))],
        out_specs=pl.BlockSpec((BLK // 2, 2 * D), lambda b: (b, 0)),
        out_shape=jax.ShapeDtypeStruct((nblk * BLK // 2, 2 * D), jnp.float32),
    )(xt)
    return packed.reshape(nblk * BLK, D)


def _sc_scores(u_rm, v_rm, uidx, vidx, B, NW, NCHUNK, CH, S, D):
    BW = B // NW
    mesh = plsc.VectorSubcoreMesh(core_axis_name="c", subcore_axis_name="s")
    NC = 2

    @functools.partial(
        pl.kernel,
        mesh=mesh,
        compiler_params=pltpu.CompilerParams(
            needs_layout_passes=False, use_tc_tiling_on_sc=False),
        out_type=[
            jax.ShapeDtypeStruct((B,), jnp.float32),
            jax.ShapeDtypeStruct((B,), jnp.float32),
        ],
        scratch_types=[
            pltpu.VMEM((NCHUNK, CH), jnp.int32),
            pltpu.VMEM((S, NCHUNK, CH), jnp.int32),
            pltpu.VMEM((CH, D), jnp.float32),
            pltpu.VMEM((S, CH, D), jnp.float32),
            pltpu.VMEM((BW,), jnp.float32),
            pltpu.VMEM((BW,), jnp.float32),
            pltpu.SemaphoreType.DMA,
        ],
    )
    def sc_kernel(u_hbm, v_hbm, uidx_hbm, vidx_hbm, pos_hbm, neg_hbm,
                  uidx_v, vidx_v, ubuf, vbuf, pos_v, neg_v, sem):
        wid = lax.axis_index("s") * NC + lax.axis_index("c")
        pltpu.sync_copy(uidx_hbm.at[wid], uidx_v)
        pltpu.sync_copy(vidx_hbm.at[wid], vidx_v)
        lane = lax.broadcasted_iota(jnp.int32, (L,), 0)

        def do_chunk(c, carry):
            cps = [pltpu.async_copy(u_hbm.at[uidx_v.at[c]], ubuf, sem)]
            for s in range(S):
                cps.append(
                    pltpu.async_copy(v_hbm.at[vidx_v.at[s, c]], vbuf.at[s], sem))
            for cp in cps:
                cp.wait()
            UNROLL = 4
            for g in range(CH // L):
                rows = g * L + lane

                def dot_body(d0, acc):
                    ap, an = acc
                    loads = []
                    for du in range(UNROLL):
                        col = jnp.broadcast_to(d0 * UNROLL + du, (L,))
                        uu = plsc.load_gather(ubuf, [rows, col])
                        vs = [plsc.load_gather(
                                  vbuf, [jnp.full((L,), s, jnp.int32), rows, col])
                              for s in range(S)]
                        loads.append((uu, vs))
                    for uu, vs in loads:
                        ns = (vs[1] + vs[2]) + (vs[3] + vs[4]) + vs[5]
                        ap = ap + uu * vs[0]
                        an = an + uu * ns
                    return ap, an

                zero = jnp.zeros((L,), jnp.float32)
                ap, an = lax.fori_loop(0, D // UNROLL, dot_body, (zero, zero))
                pos_v[pl.ds(c * CH + g * L, L)] = ap
                neg_v[pl.ds(c * CH + g * L, L)] = an
            return carry

        lax.fori_loop(0, NCHUNK, do_chunk, 0)
        base = wid * BW
        pltpu.sync_copy(pos_v, pos_hbm.at[pl.ds(base, BW)])
        pltpu.sync_copy(neg_v, neg_hbm.at[pl.ds(base, BW)])

    return sc_kernel(u_rm, v_rm, uidx, vidx)


def _tc_loss(pos, neg, bs):
    def body(bs_ref, p_ref, n_ref, o_ref):
        def logsig(t):
            return jnp.minimum(t, 0.0) - jnp.log1p(jnp.exp(-jnp.abs(t)))

        tot = jnp.sum(logsig(p_ref[...]) + logsig(-n_ref[...]))
        o_ref[0, 0] = -tot / bs_ref[0, 0]

    out = pl.pallas_call(
        body,
        out_shape=jax.ShapeDtypeStruct((1, 1), jnp.float32),
        in_specs=[
            pl.BlockSpec(memory_space=pltpu.SMEM),
            pl.BlockSpec(memory_space=pltpu.VMEM),
            pl.BlockSpec(memory_space=pltpu.VMEM),
        ],
        out_specs=pl.BlockSpec(memory_space=pltpu.SMEM),
    )(bs, pos, neg)
    return out[0, 0]


def kernel(u_pos, v_pos, v_neg, batch_size, u_emb, v_emb):
    B = u_pos.shape[0]
    NNEG = v_neg.shape[1]
    V, D = u_emb.shape
    S = 1 + NNEG
    NW = 32
    BW = B // NW
    CH = 128
    NCHUNK = BW // CH

    BLK = 16384
    u_rm = _tc_transpose(u_emb.T, V, D, BLK)
    v_rm = _tc_transpose(v_emb.T, V, D, BLK)

    def remap(v):
        h = BLK // 2
        return 2 * ((v // BLK) * h + v % h) + (v % BLK) // h

    uidx = remap(u_pos).reshape(NW, NCHUNK, CH)
    vidx = remap(jnp.concatenate([v_pos[None, :], v_neg.T], axis=0))
    vidx = vidx.reshape(S, NW, NCHUNK, CH).transpose(1, 0, 2, 3)

    pos, neg = _sc_scores(u_rm, v_rm, uidx, vidx, B, NW, NCHUNK, CH, S, D)

    r = B // 128
    bs = jnp.asarray(batch_size, jnp.float32).reshape(1, 1)
    return _tc_loss(pos.reshape(r, 128), neg.reshape(r, 128), bs)


# speedup vs baseline: 1.0021x; 1.0021x over previous
import functools

import jax
import jax.numpy as jnp
from jax import lax
from jax.experimental import pallas as pl
from jax.experimental.pallas import tpu as pltpu
from jax.experimental.pallas import tpu_sc as plsc

L = 16


def _tc_transpose(xt, V, D, BLK):
    def body(x_ref, o_ref):
        eye = jnp.eye(D, dtype=jnp.float32)
        h = BLK // 2
        e = jax.lax.dot_general(
            x_ref[:, :h], eye, (((0,), (0,)), ((), ())),
            preferred_element_type=jnp.float32)
        o = jax.lax.dot_general(
            x_ref[:, h:], eye, (((0,), (0,)), ((), ())),
            preferred_element_type=jnp.float32)
        o_ref[...] = jnp.concatenate([e, o], axis=1)

    nblk = pl.cdiv(V, BLK)
    packed = pl.pallas_call(
        body,
        grid=(nblk,),
        in_specs=[pl.BlockSpec((D, BLK), lambda b: (0, b))],
        out_specs=pl.BlockSpec((BLK // 2, 2 * D), lambda b: (b, 0)),
        out_shape=jax.ShapeDtypeStruct((nblk * BLK // 2, 2 * D), jnp.float32),
    )(xt)
    return packed.reshape(nblk * BLK, D)


def _sc_scores(u_rm, v_rm, uidx, vidx, B, NW, NCHUNK, CH, S, D):
    BW = B // NW
    mesh = plsc.VectorSubcoreMesh(core_axis_name="c", subcore_axis_name="s")
    NC = 2

    @functools.partial(
        pl.kernel,
        mesh=mesh,
        compiler_params=pltpu.CompilerParams(
            needs_layout_passes=False, use_tc_tiling_on_sc=False),
        out_type=[
            jax.ShapeDtypeStruct((B,), jnp.float32),
            jax.ShapeDtypeStruct((B,), jnp.float32),
        ],
        scratch_types=[
            pltpu.VMEM((NCHUNK, CH), jnp.int32),
            pltpu.VMEM((S, NCHUNK, CH), jnp.int32),
            pltpu.VMEM((CH, D), jnp.float32),
            pltpu.VMEM((S, CH, D), jnp.float32),
            pltpu.VMEM((BW,), jnp.float32),
            pltpu.VMEM((BW,), jnp.float32),
            pltpu.SemaphoreType.DMA,
        ],
    )
    def sc_kernel(u_hbm, v_hbm, uidx_hbm, vidx_hbm, pos_hbm, neg_hbm,
                  uidx_v, vidx_v, ubuf, vbuf, pos_v, neg_v, sem):
        wid = lax.axis_index("s") * NC + lax.axis_index("c")
        pltpu.sync_copy(uidx_hbm.at[wid], uidx_v)
        pltpu.sync_copy(vidx_hbm.at[wid], vidx_v)
        lane = lax.broadcasted_iota(jnp.int32, (L,), 0)

        def do_chunk(c, carry):
            cps = [pltpu.async_copy(u_hbm.at[uidx_v.at[c]], ubuf, sem)]
            for s in range(S):
                cps.append(
                    pltpu.async_copy(v_hbm.at[vidx_v.at[s, c]], vbuf.at[s], sem))
            for cp in cps:
                cp.wait()
            UNROLL = 4
            for g in range(CH // L):
                rows = g * L + lane

                def dot_body(d0, acc):
                    aps, ans = acc
                    loads = []
                    for du in range(UNROLL):
                        col = jnp.broadcast_to(d0 * UNROLL + du, (L,))
                        uu = plsc.load_gather(ubuf, [rows, col])
                        vs = [plsc.load_gather(
                                  vbuf, [jnp.full((L,), s, jnp.int32), rows, col])
                              for s in range(S)]
                        loads.append((uu, vs))
                    aps = list(aps)
                    ans = list(ans)
                    for du, (uu, vs) in enumerate(loads):
                        ns = (vs[1] + vs[2]) + (vs[3] + vs[4]) + vs[5]
                        aps[du] = aps[du] + uu * vs[0]
                        ans[du] = ans[du] + uu * ns
                    return tuple(aps), tuple(ans)

                zero = jnp.zeros((L,), jnp.float32)
                zeros = (zero,) * UNROLL
                aps, ans = lax.fori_loop(0, D // UNROLL, dot_body, (zeros, zeros))
                ap = (aps[0] + aps[1]) + (aps[2] + aps[3])
                an = (ans[0] + ans[1]) + (ans[2] + ans[3])
                pos_v[pl.ds(c * CH + g * L, L)] = ap
                neg_v[pl.ds(c * CH + g * L, L)] = an
            return carry

        lax.fori_loop(0, NCHUNK, do_chunk, 0)
        base = wid * BW
        pltpu.sync_copy(pos_v, pos_hbm.at[pl.ds(base, BW)])
        pltpu.sync_copy(neg_v, neg_hbm.at[pl.ds(base, BW)])

    return sc_kernel(u_rm, v_rm, uidx, vidx)


def _tc_loss(pos, neg, bs):
    def body(bs_ref, p_ref, n_ref, o_ref):
        def logsig(t):
            return jnp.minimum(t, 0.0) - jnp.log1p(jnp.exp(-jnp.abs(t)))

        tot = jnp.sum(logsig(p_ref[...]) + logsig(-n_ref[...]))
        o_ref[0, 0] = -tot / bs_ref[0, 0]

    out = pl.pallas_call(
        body,
        out_shape=jax.ShapeDtypeStruct((1, 1), jnp.float32),
        in_specs=[
            pl.BlockSpec(memory_space=pltpu.SMEM),
            pl.BlockSpec(memory_space=pltpu.VMEM),
            pl.BlockSpec(memory_space=pltpu.VMEM),
        ],
        out_specs=pl.BlockSpec(memory_space=pltpu.SMEM),
    )(bs, pos, neg)
    return out[0, 0]


def kernel(u_pos, v_pos, v_neg, batch_size, u_emb, v_emb):
    B = u_pos.shape[0]
    NNEG = v_neg.shape[1]
    V, D = u_emb.shape
    S = 1 + NNEG
    NW = 32
    BW = B // NW
    CH = 128
    NCHUNK = BW // CH

    BLK = 16384
    u_rm = _tc_transpose(u_emb.T, V, D, BLK)
    v_rm = _tc_transpose(v_emb.T, V, D, BLK)

    def remap(v):
        h = BLK // 2
        return 2 * ((v // BLK) * h + v % h) + (v % BLK) // h

    uidx = remap(u_pos).reshape(NW, NCHUNK, CH)
    vidx = remap(jnp.concatenate([v_pos[None, :], v_neg.T], axis=0))
    vidx = vidx.reshape(S, NW, NCHUNK, CH).transpose(1, 0, 2, 3)

    pos, neg = _sc_scores(u_rm, v_rm, uidx, vidx, B, NW, NCHUNK, CH, S, D)

    r = B // 128
    bs = jnp.asarray(batch_size, jnp.float32).reshape(1, 1)
    return _tc_loss(pos.reshape(r, 128), neg.reshape(r, 128), bs)


# speedup vs baseline: 1.2104x; 1.2078x over previous
import functools

import jax
import jax.numpy as jnp
from jax import lax
from jax.experimental import pallas as pl
from jax.experimental.pallas import tpu as pltpu
from jax.experimental.pallas import tpu_sc as plsc

L = 16


def _tc_transpose(xt, V, D, BLK):
    def body(x_ref, o_ref):
        eye = jnp.eye(D, dtype=jnp.float32)
        h = BLK // 2
        e = jax.lax.dot_general(
            x_ref[:, :h], eye, (((0,), (0,)), ((), ())),
            preferred_element_type=jnp.float32)
        o = jax.lax.dot_general(
            x_ref[:, h:], eye, (((0,), (0,)), ((), ())),
            preferred_element_type=jnp.float32)
        o_ref[...] = jnp.concatenate([e, o], axis=1)

    nblk = pl.cdiv(V, BLK)
    packed = pl.pallas_call(
        body,
        grid=(nblk,),
        in_specs=[pl.BlockSpec((D, BLK), lambda b: (0, b))],
        out_specs=pl.BlockSpec((BLK // 2, 2 * D), lambda b: (b, 0)),
        out_shape=jax.ShapeDtypeStruct((nblk * BLK // 2, 2 * D), jnp.float32),
    )(xt)
    return packed.reshape(nblk * BLK, D)


def _sc_scores(u_rm, v_rm, uidx, vidx, B, NW, NCHUNK, CH, S, D):
    BW = B // NW
    mesh = plsc.VectorSubcoreMesh(core_axis_name="c", subcore_axis_name="s")
    NC = 2

    @functools.partial(
        pl.kernel,
        mesh=mesh,
        compiler_params=pltpu.CompilerParams(
            needs_layout_passes=False, use_tc_tiling_on_sc=False),
        out_type=[
            jax.ShapeDtypeStruct((B,), jnp.float32),
            jax.ShapeDtypeStruct((B,), jnp.float32),
        ],
        scratch_types=[
            pltpu.VMEM((NCHUNK, CH), jnp.int32),
            pltpu.VMEM((S, NCHUNK, CH), jnp.int32),
            pltpu.VMEM((CH, D), jnp.float32),
            pltpu.VMEM((S, CH, D), jnp.float32),
            pltpu.VMEM((BW,), jnp.float32),
            pltpu.VMEM((BW,), jnp.float32),
            pltpu.SemaphoreType.DMA,
        ],
    )
    def sc_kernel(u_hbm, v_hbm, uidx_hbm, vidx_hbm, pos_hbm, neg_hbm,
                  uidx_v, vidx_v, ubuf, vbuf, pos_v, neg_v, sem):
        wid = lax.axis_index("s") * NC + lax.axis_index("c")
        pltpu.sync_copy(uidx_hbm.at[wid], uidx_v)
        pltpu.sync_copy(vidx_hbm.at[wid], vidx_v)
        lane = lax.broadcasted_iota(jnp.int32, (L,), 0)

        def do_chunk(c, carry):
            cps = [pltpu.async_copy(u_hbm.at[uidx_v.at[c]], ubuf, sem)]
            for s in range(S):
                cps.append(
                    pltpu.async_copy(v_hbm.at[vidx_v.at[s, c]], vbuf.at[s], sem))
            for cp in cps:
                cp.wait()
            UNROLL = 4
            for g in range(CH // L):
                rows = g * L + lane

                def dot_body(d0, acc):
                    aps, ans = acc
                    loads = []
                    for du in range(UNROLL):
                        col = (jnp.broadcast_to(d0 * UNROLL + du, (L,)) + lane) & (D - 1)
                        uu = plsc.load_gather(ubuf, [rows, col])
                        vs = [plsc.load_gather(
                                  vbuf, [jnp.full((L,), s, jnp.int32), rows, col])
                              for s in range(S)]
                        loads.append((uu, vs))
                    aps = list(aps)
                    ans = list(ans)
                    for du, (uu, vs) in enumerate(loads):
                        ns = (vs[1] + vs[2]) + (vs[3] + vs[4]) + vs[5]
                        aps[du] = aps[du] + uu * vs[0]
                        ans[du] = ans[du] + uu * ns
                    return tuple(aps), tuple(ans)

                zero = jnp.zeros((L,), jnp.float32)
                zeros = (zero,) * UNROLL
                aps, ans = lax.fori_loop(0, D // UNROLL, dot_body, (zeros, zeros))
                ap = (aps[0] + aps[1]) + (aps[2] + aps[3])
                an = (ans[0] + ans[1]) + (ans[2] + ans[3])
                pos_v[pl.ds(c * CH + g * L, L)] = ap
                neg_v[pl.ds(c * CH + g * L, L)] = an
            return carry

        lax.fori_loop(0, NCHUNK, do_chunk, 0)
        base = wid * BW
        pltpu.sync_copy(pos_v, pos_hbm.at[pl.ds(base, BW)])
        pltpu.sync_copy(neg_v, neg_hbm.at[pl.ds(base, BW)])

    return sc_kernel(u_rm, v_rm, uidx, vidx)


def _tc_loss(pos, neg, bs):
    def body(bs_ref, p_ref, n_ref, o_ref):
        def logsig(t):
            return jnp.minimum(t, 0.0) - jnp.log1p(jnp.exp(-jnp.abs(t)))

        tot = jnp.sum(logsig(p_ref[...]) + logsig(-n_ref[...]))
        o_ref[0, 0] = -tot / bs_ref[0, 0]

    out = pl.pallas_call(
        body,
        out_shape=jax.ShapeDtypeStruct((1, 1), jnp.float32),
        in_specs=[
            pl.BlockSpec(memory_space=pltpu.SMEM),
            pl.BlockSpec(memory_space=pltpu.VMEM),
            pl.BlockSpec(memory_space=pltpu.VMEM),
        ],
        out_specs=pl.BlockSpec(memory_space=pltpu.SMEM),
    )(bs, pos, neg)
    return out[0, 0]


def kernel(u_pos, v_pos, v_neg, batch_size, u_emb, v_emb):
    B = u_pos.shape[0]
    NNEG = v_neg.shape[1]
    V, D = u_emb.shape
    S = 1 + NNEG
    NW = 32
    BW = B // NW
    CH = 128
    NCHUNK = BW // CH

    BLK = 16384
    u_rm = _tc_transpose(u_emb.T, V, D, BLK)
    v_rm = _tc_transpose(v_emb.T, V, D, BLK)

    def remap(v):
        h = BLK // 2
        return 2 * ((v // BLK) * h + v % h) + (v % BLK) // h

    uidx = remap(u_pos).reshape(NW, NCHUNK, CH)
    vidx = remap(jnp.concatenate([v_pos[None, :], v_neg.T], axis=0))
    vidx = vidx.reshape(S, NW, NCHUNK, CH).transpose(1, 0, 2, 3)

    pos, neg = _sc_scores(u_rm, v_rm, uidx, vidx, B, NW, NCHUNK, CH, S, D)

    r = B // 128
    bs = jnp.asarray(batch_size, jnp.float32).reshape(1, 1)
    return _tc_loss(pos.reshape(r, 128), neg.reshape(r, 128), bs)
